# baseline (speedup 1.0000x reference)
import functools

import jax
import jax.numpy as jnp
from jax.experimental import pallas as pl
from jax.experimental.pallas import tpu as pltpu

_BIG = 1e30


def _chamfer_body(p8_ref, g8t_ref, m_ref, c_ref, out_p_ref, out_g_ref,
                  colmin_ref, *, num_i):
    i = pl.program_id(1)
    b = pl.program_id(0)

    P = p8_ref[0]
    G = g8t_ref[0]
    m = m_ref[0]
    conf = c_ref[0]

    pn = jnp.sum(P * P, axis=1, keepdims=True)
    gn = jnp.sum(G * G, axis=0, keepdims=True)
    pnb = pn + (1.0 - m) * _BIG

    A = jnp.concatenate([P[:, :3] * (-2.0), pnb,
                         jnp.ones((P.shape[0], 1), jnp.float32)], axis=1)
    GA = jnp.concatenate([G[:3, :], jnp.ones((1, G.shape[1]), jnp.float32),
                          gn], axis=0)
    D = jax.lax.dot_general(A, GA, (((1,), (0,)), ((), ())),
                            preferred_element_type=jnp.float32)

    rowmin = jnp.min(D, axis=1, keepdims=True)
    safe_conf = jnp.where(m > 0, conf, 1.0)
    step_p = jnp.sum(jnp.maximum(rowmin, 0.0) * conf * m
                     - jnp.log(safe_conf) * m)

    cmin = jnp.min(D, axis=0, keepdims=True)

    @pl.when(i == 0)
    def _():
        colmin_ref[...] = cmin

    @pl.when(i > 0)
    def _():
        colmin_ref[...] = jnp.minimum(colmin_ref[...], cmin)

    @pl.when((i == 0) & (b == 0))
    def _():
        out_p_ref[...] = jnp.zeros_like(out_p_ref)
        out_g_ref[...] = jnp.zeros_like(out_g_ref)

    out_p_ref[...] += jnp.full((1, 1), step_p, jnp.float32)

    @pl.when(i == num_i - 1)
    def _():
        out_g_ref[...] += jnp.full(
            (1, 1), jnp.sum(jnp.maximum(colmin_ref[...], 0.0)), jnp.float32)


def kernel(x_gt, x_pred, mask, confidence):
    B, V1, _ = x_gt.shape
    V2 = x_pred.shape[1]
    TI = 512
    num_i = V2 // TI

    zeros_p = jnp.zeros((B, V2, 5), jnp.float32)
    p8 = jnp.concatenate([x_pred, zeros_p], axis=-1)
    g8t = jnp.swapaxes(
        jnp.concatenate([x_gt, jnp.zeros((B, V1, 5), jnp.float32)], axis=-1),
        1, 2)
    m3 = mask.astype(jnp.float32)
    c3 = confidence[..., None]

    out_p, out_g = pl.pallas_call(
        functools.partial(_chamfer_body, num_i=num_i),
        grid=(B, num_i),
        in_specs=[
            pl.BlockSpec((1, TI, 8), lambda b, i: (b, i, 0)),
            pl.BlockSpec((1, 8, V1), lambda b, i: (b, 0, 0)),
            pl.BlockSpec((1, TI, 1), lambda b, i: (b, i, 0)),
            pl.BlockSpec((1, TI, 1), lambda b, i: (b, i, 0)),
        ],
        out_specs=[
            pl.BlockSpec((1, 1), lambda b, i: (0, 0)),
            pl.BlockSpec((1, 1), lambda b, i: (0, 0)),
        ],
        out_shape=[
            jax.ShapeDtypeStruct((1, 1), jnp.float32),
            jax.ShapeDtypeStruct((1, 1), jnp.float32),
        ],
        scratch_shapes=[pltpu.VMEM((1, V1), jnp.float32)],
    )(p8, g8t, m3, c3)

    return out_p[0, 0] / (B * V2) + out_g[0, 0] / (B * V1)


# speedup vs baseline: 1.1614x; 1.1614x over previous
import functools

import jax
import jax.numpy as jnp
from jax.experimental import pallas as pl
from jax.experimental.pallas import tpu as pltpu

_BIG = 1e30


def _chamfer_body(p8_ref, g8t_ref, m_ref, c_ref, out_p_ref, out_g_ref,
                  colmin_ref, *, num_i):
    i = pl.program_id(1)
    b = pl.program_id(0)

    P = p8_ref[0]
    G = g8t_ref[0]
    m = m_ref[0]
    conf = c_ref[0]

    pn = jnp.sum(P * P, axis=1, keepdims=True)
    gn = jnp.sum(G * G, axis=0, keepdims=True)
    pnb = pn + (1.0 - m) * _BIG

    E = jax.lax.dot_general(P * (-2.0), G, (((1,), (0,)), ((), ())),
                            preferred_element_type=jnp.float32)
    D = E + pnb + gn

    rowmin = jnp.min(D, axis=1, keepdims=True)
    safe_conf = jnp.where(m > 0, conf, 1.0)
    step_p = jnp.sum(jnp.maximum(rowmin, 0.0) * conf * m
                     - jnp.log(safe_conf) * m)

    cmin = jnp.min(D, axis=0, keepdims=True)

    @pl.when(i == 0)
    def _():
        colmin_ref[...] = cmin

    @pl.when(i > 0)
    def _():
        colmin_ref[...] = jnp.minimum(colmin_ref[...], cmin)

    @pl.when((i == 0) & (b == 0))
    def _():
        out_p_ref[...] = jnp.zeros_like(out_p_ref)
        out_g_ref[...] = jnp.zeros_like(out_g_ref)

    out_p_ref[...] += jnp.full((1, 1), step_p, jnp.float32)

    @pl.when(i == num_i - 1)
    def _():
        out_g_ref[...] += jnp.full(
            (1, 1), jnp.sum(jnp.maximum(colmin_ref[...], 0.0)), jnp.float32)


def kernel(x_gt, x_pred, mask, confidence):
    B, V1, _ = x_gt.shape
    V2 = x_pred.shape[1]
    TI = 1024
    num_i = V2 // TI

    zeros_p = jnp.zeros((B, V2, 5), jnp.float32)
    p8 = jnp.concatenate([x_pred, zeros_p], axis=-1)
    g8t = jnp.swapaxes(
        jnp.concatenate([x_gt, jnp.zeros((B, V1, 5), jnp.float32)], axis=-1),
        1, 2)
    m3 = mask.astype(jnp.float32)
    c3 = confidence[..., None]

    out_p, out_g = pl.pallas_call(
        functools.partial(_chamfer_body, num_i=num_i),
        grid=(B, num_i),
        in_specs=[
            pl.BlockSpec((1, TI, 8), lambda b, i: (b, i, 0)),
            pl.BlockSpec((1, 8, V1), lambda b, i: (b, 0, 0)),
            pl.BlockSpec((1, TI, 1), lambda b, i: (b, i, 0)),
            pl.BlockSpec((1, TI, 1), lambda b, i: (b, i, 0)),
        ],
        out_specs=[
            pl.BlockSpec((1, 1), lambda b, i: (0, 0)),
            pl.BlockSpec((1, 1), lambda b, i: (0, 0)),
        ],
        out_shape=[
            jax.ShapeDtypeStruct((1, 1), jnp.float32),
            jax.ShapeDtypeStruct((1, 1), jnp.float32),
        ],
        scratch_shapes=[pltpu.VMEM((1, V1), jnp.float32)],
    )(p8, g8t, m3, c3)

    return out_p[0, 0] / (B * V2) + out_g[0, 0] / (B * V1)
